# Initial kernel scaffold; baseline (speedup 1.0000x reference)
import jax
import jax.numpy as jnp
from jax.experimental import pallas as pl


def kernel(mesh_x, obj_x, mesh_kin, obj_kin, mm_index, mo_index, om_index, ff_index, e_mm, e_mo, e_om, e_ff, params):
    raise NotImplementedError("write your pallas kernel here")


# speedup vs baseline: 1.5127x; 1.5127x over previous
import functools

import jax
import jax.numpy as jnp
from jax import lax
from jax.experimental import pallas as pl
from jax.experimental.pallas import tpu as pltpu
from jax.experimental.pallas import tpu_sc as plsc

L = 128
BLK = 512

N_MESH = 50000
N_OBJ = 10000
NPAD_M = 51200
NPAD_O = 12288
EPAD = {"mm": 802816, "mo": 81920, "om": 81920, "ff": 114688}

NW = 32
GB = 128


def _ln(y, g, b):
    mu = jnp.mean(y, axis=-1, keepdims=True)
    yc = y - mu
    var = jnp.mean(yc * yc, axis=-1, keepdims=True)
    return yc * lax.rsqrt(var + 1e-5) * g + b


def _enc_node_body(x_ref, a_ref, k_ref, w2_ref, vec_ref, o_ref):
    xv = x_ref[...]
    x = xv[:, :8]
    kin = xv[:, 8:9]
    kv = k_ref[...]
    krow = jnp.where(kin < 0.5, kv[0:1], jnp.where(kin < 1.5, kv[1:2], kv[2:3]))
    h = jnp.maximum(jnp.dot(x, a_ref[...], preferred_element_type=jnp.float32)
                    + krow + vec_ref[0], 0.0)
    y = jnp.dot(h, w2_ref[...], preferred_element_type=jnp.float32) + vec_ref[1]
    o_ref[...] = _ln(y, vec_ref[2], vec_ref[3])


def _enc_edge_body(x_ref, a_ref, w2_ref, vec_ref, o_ref):
    h = jnp.maximum(jnp.dot(x_ref[...], a_ref[...], preferred_element_type=jnp.float32)
                    + vec_ref[0], 0.0)
    y = jnp.dot(h, w2_ref[...], preferred_element_type=jnp.float32) + vec_ref[1]
    o_ref[...] = _ln(y, vec_ref[2], vec_ref[3])


def _edge_upd_body(g_ref, l_ref, w3_ref, w4_ref, zu_ref, vec_ref, o_ref, z_ref):
    lv = l_ref[...]
    h = jnp.maximum(jnp.dot(lv, w3_ref[...], preferred_element_type=jnp.float32)
                    + g_ref[...] + vec_ref[0], 0.0)
    y = jnp.dot(h, w4_ref[...], preferred_element_type=jnp.float32) + vec_ref[1]
    lnew = lv + _ln(y, vec_ref[2], vec_ref[3])
    o_ref[...] = lnew
    z_ref[...] = jnp.dot(lnew, zu_ref[...], preferred_element_type=jnp.float32)


def _node_updp_body(m_ref, a_ref, u1_ref, w2_ref, vec_ref, o_ref):
    mv = m_ref[...]
    h = jnp.maximum(jnp.dot(mv, u1_ref[...], preferred_element_type=jnp.float32)
                    + a_ref[...] + vec_ref[0], 0.0)
    y = jnp.dot(h, w2_ref[...], preferred_element_type=jnp.float32) + vec_ref[1]
    o_ref[...] = mv + _ln(y, vec_ref[2], vec_ref[3])


def _dec_body(m_ref, w1_ref, w2_ref, vec_ref, o_ref):
    h = jnp.maximum(jnp.dot(m_ref[...], w1_ref[...], preferred_element_type=jnp.float32)
                    + vec_ref[0], 0.0)
    o_ref[...] = jnp.dot(h, w2_ref[...], preferred_element_type=jnp.float32) + vec_ref[1]


def _proj_body(x_ref, w_ref, *o_refs):
    y = jnp.dot(x_ref[...], w_ref[...], preferred_element_type=jnp.float32)
    for i, o in enumerate(o_refs):
        o[...] = y[:, i * L:(i + 1) * L]


def _rows_spec(blk, d):
    return pl.BlockSpec((blk, d), lambda i: (i, 0))


def _full_spec(shape):
    nd = len(shape)
    return pl.BlockSpec(shape, lambda i: (0,) * nd)


def _tc_call(body, row_ins, aux_ins, n_out=1, out_dim=L, out_chunks=None):
    n = row_ins[0].shape[-2]
    grid = (n // BLK,)
    in_specs = [
        pl.BlockSpec((x.shape[0], BLK, x.shape[2]), lambda i: (0, i, 0))
        if x.ndim == 3 else _rows_spec(BLK, x.shape[1])
        for x in row_ins]
    in_specs += [_full_spec(x.shape) for x in aux_ins]
    if out_chunks is not None:
        out_specs = pl.BlockSpec((out_chunks, BLK, L // out_chunks),
                                 lambda i: (0, i, 0))
        out_shape = jax.ShapeDtypeStruct((out_chunks, n, L // out_chunks),
                                         jnp.float32)
    elif n_out == 1:
        out_specs = _rows_spec(BLK, out_dim)
        out_shape = jax.ShapeDtypeStruct((n, out_dim), jnp.float32)
    else:
        out_specs = [_rows_spec(BLK, out_dim) for _ in range(n_out)]
        out_shape = [jax.ShapeDtypeStruct((n, out_dim), jnp.float32)
                     for _ in range(n_out)]
    return pl.pallas_call(
        body, grid=grid, in_specs=in_specs, out_specs=out_specs,
        out_shape=out_shape,
    )(*row_ins, *aux_ins)


@functools.lru_cache(maxsize=None)
def _make_sc_gather2(n1, n2, e):
    eperw = e // NW
    nblk = eperw // GB
    mesh = plsc.VectorSubcoreMesh(core_axis_name="c", subcore_axis_name="s")

    @functools.partial(
        pl.kernel, mesh=mesh,
        out_type=jax.ShapeDtypeStruct((e, L), jnp.float32),
        scratch_types=[
            pltpu.VMEM((eperw,), jnp.int32),
            pltpu.VMEM((eperw,), jnp.int32),
            pltpu.VMEM((GB, L), jnp.float32),
            pltpu.VMEM((GB, L), jnp.float32),
            pltpu.SemaphoreType.DMA,
            pltpu.SemaphoreType.DMA,
        ])
    def k(p1, p2, i1, i2, out, i1v, i2v, r1, r2, s1, s2):
        wid = lax.axis_index("s") * 2 + lax.axis_index("c")
        base = wid * eperw
        pltpu.sync_copy(i1.at[pl.ds(base, eperw)], i1v)
        pltpu.sync_copy(i2.at[pl.ds(base, eperw)], i2v)

        def body(j, _):
            a = pltpu.async_copy(p1.at[i1v.at[pl.ds(j * GB, GB)]], r1, s1)
            b = pltpu.async_copy(p2.at[i2v.at[pl.ds(j * GB, GB)]], r2, s2)
            a.wait()
            b.wait()

            def add_row(r, _):
                for c in range(L // 16):
                    sl = pl.ds(c * 16, 16)
                    r1[r, sl] = r1[r, sl] + r2[r, sl]
                return 0

            lax.fori_loop(0, GB, add_row, 0)
            pltpu.sync_copy(r1, out.at[pl.ds(base + j * GB, GB)])
            return 0

        lax.fori_loop(0, nblk, body, 0)

    return k


def _segsum_kernel_body(sets, bounds, agg, idxw, rbuf, acc, nout, rng, ndata):
    cid = lax.axis_index("c")
    sid = lax.axis_index("s")
    lanes = lax.iota(jnp.int32, 16)
    zeros = jnp.zeros((16,), jnp.float32)
    iv = idxw
    accn = rng + 128

    def rr_copy(nblocks, body):
        def go(t, _):
            body(sid + t * 16)
            return 0
        lax.fori_loop(0, (nblocks - sid + 15) // 16, go, 0)

    def zero_rbuf(r, _):
        for c in range(8):
            rbuf[r, pl.ds(c * 16, 16)] = zeros
        return 0

    def run_range(r):
        r0 = r * rng
        lax.fori_loop(0, 128, zero_rbuf, 0)
        rr_copy(accn // 128,
                lambda b: pltpu.sync_copy(rbuf, acc.at[pl.ds(b * 128, 128)]))
        plsc.subcore_barrier()

        for (l_ref, dd, e), bound in zip(sets, bounds):
            epert = e // 16
            ngrp = epert // 1024

            def egroup(gi, _, l_ref=l_ref, dd=dd, epert=epert):
                pltpu.sync_copy(dd.at[sid, pl.ds(gi * 8, 8)], iv.at[pl.ds(1, 8)])
                for j8 in range(8):
                    pltpu.sync_copy(
                        l_ref.at[pl.ds(sid * epert + gi * 1024 + j8 * 128, 128)],
                        rbuf)
                    for c in range(8):
                        sl = pl.ds(c * 16, 16)
                        t = iv[1 + j8, sl] - r0
                        ok = (t >= 0) & (t < rng)
                        idxw[0, sl] = jnp.where(ok, t, rng + lanes)
                    pltpu.sync_copy(rbuf, acc.at[idxw.at[0]], add=True)
                return 0

            if bound <= rng:
                @pl.when(r0 == 0)
                def _():
                    lax.fori_loop(0, ngrp, egroup, 0)
            else:
                lax.fori_loop(0, ngrp, egroup, 0)
        plsc.subcore_barrier()

        def ob(b):
            pltpu.sync_copy(acc.at[pl.ds(b * 128, 128)], rbuf)
            pltpu.sync_copy(rbuf, agg.at[pl.ds(r0 + b * 128, 128)])

        rr_copy(rng // 128, ob)
        plsc.subcore_barrier()

    for ch in range((ndata + 1) // 2):
        r = cid + 2 * ch
        if 2 * ch + 1 < ndata:
            run_range(r)
        else:
            @pl.when(cid == 0)
            def _():
                run_range(r)


@functools.lru_cache(maxsize=None)
def _make_sc_segsum3(es, bounds, nout, rng, ndata):
    e0, e1, e2 = es
    mesh = plsc.VectorSubcoreMesh(core_axis_name="c", subcore_axis_name="s")

    @functools.partial(
        pl.kernel, mesh=mesh,
        out_type=jax.ShapeDtypeStruct((nout, L), jnp.float32),
        scratch_types=[
            pltpu.VMEM((9, 128), jnp.int32),
            pltpu.VMEM((128, L), jnp.float32),
            pltpu.VMEM_SHARED((rng + 128, L), jnp.float32),
        ])
    def k(l0, d0, l1, d1, l2, d2, agg, idxw, rbuf, acc):
        _segsum_kernel_body([(l0, d0, e0), (l1, d1, e1), (l2, d2, e2)],
                            bounds, agg, idxw, rbuf, acc, nout, rng, ndata)

    return k


@functools.lru_cache(maxsize=None)
def _make_sc_segsum1(e0, bound, nout, rng, ndata):
    mesh = plsc.VectorSubcoreMesh(core_axis_name="c", subcore_axis_name="s")

    @functools.partial(
        pl.kernel, mesh=mesh,
        out_type=jax.ShapeDtypeStruct((nout, L), jnp.float32),
        scratch_types=[
            pltpu.VMEM((9, 128), jnp.int32),
            pltpu.VMEM((128, L), jnp.float32),
            pltpu.VMEM_SHARED((rng + 128, L), jnp.float32),
        ])
    def k(l0, d0, agg, idxw, rbuf, acc):
        _segsum_kernel_body([(l0, d0, e0)], (bound,), agg, idxw, rbuf, acc,
                            nout, rng, ndata)

    return k


def _sc_gather2(p1, p2, i1, i2):
    return _make_sc_gather2(p1.shape[0], p2.shape[0], i1.shape[0])(p1, p2, i1, i2)


def _sc_segsum_mesh(zmm, dmm, zff, dff, zom, dom):
    k = _make_sc_segsum3((zmm.shape[0], zff.shape[0], zom.shape[0]),
                         (NPAD_M, NPAD_M, 12800), NPAD_M, 12800, 4)
    return k(zmm, dmm, zff, dff, zom, dom)


def _sc_segsum_obj(zmo, dmo):
    return _make_sc_segsum1(zmo.shape[0], NPAD_O, NPAD_O, 6144, 2)(zmo, dmo)


def _vecs(p, b1):
    return jnp.stack([b1, p["b"][1], p["ln_g"], p["ln_b"]])


def _enc_node_prep(p, mean, std):
    w1 = p["W"][0]
    a = w1[:8] / std[:8, None]
    kmat = ((jnp.eye(3, dtype=jnp.float32) - mean[8:][None, :]) / std[8:][None, :]) @ w1[8:]
    b1 = p["b"][0] - (mean[:8] / std[:8]) @ w1[:8]
    return a, kmat, p["W"][1], _vecs(p, b1)


def _enc_edge_prep(p, mean, std):
    w1 = p["W"][0]
    a = w1 / std[:, None]
    b1 = p["b"][0] - (mean / std) @ w1
    return a, p["W"][1], _vecs(p, b1)


def _pad_rows(x, n):
    return jnp.pad(x, ((0, n - x.shape[0]),) + ((0, 0),) * (x.ndim - 1))


def _pad_idx(ix, n, fill):
    return jnp.pad(ix, (0, n - ix.shape[0]), constant_values=fill)


def kernel(mesh_x, obj_x, mesh_kin, obj_kin, mm_index, mo_index, om_index,
           ff_index, e_mm, e_mo, e_om, e_ff, params):
    p = params
    f32 = jnp.float32

    mx2 = jnp.concatenate(
        [_pad_rows(mesh_x, NPAD_M),
         _pad_idx(mesh_kin, NPAD_M, 0).astype(f32)[:, None]], axis=1)
    ox2 = jnp.concatenate(
        [_pad_rows(obj_x, NPAD_O),
         _pad_idx(obj_kin, NPAD_O, 0).astype(f32)[:, None]], axis=1)

    a, kmat, w2, vec = _enc_node_prep(p["mesh_enc"], p["node_mean"], p["node_std"])
    m = _tc_call(_enc_node_body, [mx2], [a, kmat, w2, vec])
    a, kmat, w2, vec = _enc_node_prep(p["obj_enc"], p["node_mean"], p["node_std"])
    o = _tc_call(_enc_node_body, [ox2], [a, kmat, w2, vec])

    lat = {}
    for name, feats, enc, mean, std in (
            ("mm", e_mm, p["mm_enc"], p["reg_mean"], p["reg_std"]),
            ("mo", e_mo, p["mo_enc"], p["reg_mean"], p["reg_std"]),
            ("om", e_om, p["om_enc"], p["reg_mean"], p["reg_std"]),
            ("ff", e_ff, p["ff_enc"], p["ff_mean"], p["ff_std"])):
        a, w2, vec = _enc_edge_prep(enc, mean, std)
        lat[name] = _tc_call(_enc_edge_body, [_pad_rows(feats, EPAD[name])],
                             [a, w2, vec])

    src = {"mm": _pad_idx(mm_index[0], EPAD["mm"], 0),
           "mo": _pad_idx(mo_index[0], EPAD["mo"], 0),
           "om": _pad_idx(om_index[0], EPAD["om"], 0),
           "ff": _pad_idx(ff_index[0], EPAD["ff"], 0)}
    dst = {"mm": _pad_idx(mm_index[1], EPAD["mm"], N_MESH),
           "mo": _pad_idx(mo_index[1], EPAD["mo"], N_OBJ),
           "om": _pad_idx(om_index[1], EPAD["om"], N_MESH),
           "ff": _pad_idx(ff_index[1], EPAD["ff"], N_MESH)}
    dst2d = {k: v.reshape(16, -1, 128) for k, v in dst.items()}

    for sp in p["steps"]:
        wmm = sp["mm_edge"]["W"][0]
        wff = sp["ff_edge"]["W"][0]
        wom = sp["om_edge"]["W"][0]
        wmo = sp["mo_edge"]["W"][0]

        pm = _tc_call(_proj_body, [m],
                      [jnp.concatenate([wmm[:L], wmm[L:2 * L], wff[:L],
                                        wff[L:2 * L], wom[L:2 * L], wmo[:L]],
                                       axis=1)], n_out=6)
        po = _tc_call(_proj_body, [o],
                      [jnp.concatenate([wom[:L], wmo[L:2 * L]], axis=1)], n_out=2)
        pmm_a, pmm_b, pff_a, pff_b, pom_b, pmo_a = pm
        pom_a, pmo_b = po

        g = {"mm": _sc_gather2(pmm_a, pmm_b, src["mm"], dst["mm"]),
             "ff": _sc_gather2(pff_a, pff_b, src["ff"], dst["ff"]),
             "om": _sc_gather2(pom_a, pom_b, src["om"], dst["om"]),
             "mo": _sc_gather2(pmo_a, pmo_b, src["mo"], dst["mo"])}

        u_mesh = sp["mesh_node"]["W"][0]
        u_obj = sp["obj_node"]["W"][0]
        zu = {"mm": u_mesh[L:2 * L], "om": u_mesh[2 * L:3 * L],
              "ff": u_mesh[3 * L:], "mo": u_obj[L:]}

        z = {}
        for name, wfull in (("mm", wmm), ("ff", wff), ("om", wom), ("mo", wmo)):
            ep = sp[name + "_edge"]
            lat[name], z[name] = _tc_call(
                _edge_upd_body, [g[name], lat[name]],
                [wfull[2 * L:], ep["W"][1], zu[name], _vecs(ep, ep["b"][0])],
                n_out=2)

        agg_m = _sc_segsum_mesh(z["mm"], dst2d["mm"], z["ff"], dst2d["ff"],
                                z["om"], dst2d["om"])
        agg_o = _sc_segsum_obj(z["mo"], dst2d["mo"])

        np_ = sp["mesh_node"]
        m = _tc_call(_node_updp_body, [m, agg_m],
                     [u_mesh[:L], np_["W"][1], _vecs(np_, np_["b"][0])])
        np_ = sp["obj_node"]
        o = _tc_call(_node_updp_body, [o, agg_o],
                     [u_obj[:L], np_["W"][1], _vecs(np_, np_["b"][0])])

    for name in ("mesh_dec", "obj_dec"):
        dp = p[name]
        w2 = jnp.zeros((L, L), f32).at[:, :3].set(dp["W"][1])
        b2 = jnp.zeros((L,), f32).at[:3].set(dp["b"][1])
        vec = jnp.stack([dp["b"][0], b2])
        if name == "mesh_dec":
            m_acc = _tc_call(_dec_body, [m], [dp["W"][0], w2, vec])
        else:
            o_acc = _tc_call(_dec_body, [o], [dp["W"][0], w2, vec])

    return (m_acc[:N_MESH, :3], o_acc[:N_OBJ, :3])
